# baseline (speedup 1.0000x reference)
import functools

import jax
import jax.numpy as jnp
from jax import lax
from jax.experimental import pallas as pl
from jax.experimental.pallas import tpu as pltpu
from jax.experimental.pallas import tpu_sc as plsc

N = 10000
NG = 64
NGT = 2 * NG
DIN = 128
DH = 64
DE = 32
E = 320000

NPAD = 10240
NT = 2 * NPAD
NSUB = 16
RSUB = NT // NSUB
DUMMY = 10200
DGW = 16
C = 128
NW = 32
ETOT = 2 * E
W = 32
NCH = ((-(-ETOT // (NW * C)) + W - 1) // W) * W
NWIN = NCH // W
EP = NW * C * NCH

RB = 1280
GRID = NT // RB

_SC_PARAMS = pltpu.CompilerParams(use_tc_tiling_on_sc=False)


def _sc_mesh():
    return plsc.VectorSubcoreMesh(core_axis_name="c", subcore_axis_name="s",
                                  num_cores=2, num_subcores=NSUB)

_HI = jax.lax.Precision.HIGHEST


def _mm(a, b):
    return jax.lax.dot_general(a, b, (((1,), (0,)), ((), ())),
                               precision=_HI,
                               preferred_element_type=jnp.float32)


def _deg_call(dst_w, zeros_nt, ones_c):
    @functools.partial(
        pl.kernel,
        out_type=jax.ShapeDtypeStruct((2, NT, DGW), jnp.float32),
        mesh=_sc_mesh(),
        compiler_params=_SC_PARAMS,
        scratch_types=[
            pltpu.VMEM((NCH, C), jnp.int32),
            pltpu.VMEM((C, DGW), jnp.float32),
            pltpu.VMEM_SHARED((NT, DGW), jnp.float32),
        ],
    )
    def deg_kernel(dst_hbm, zero_hbm, one_hbm, out_hbm, idx_v, ones_v, acc_sh):
        cid = lax.axis_index("c")
        sid = lax.axis_index("s")
        wid = cid * NSUB + sid
        r0 = sid * RSUB
        pltpu.sync_copy(zero_hbm.at[pl.ds(r0, RSUB)], acc_sh.at[pl.ds(r0, RSUB)])
        pltpu.sync_copy(one_hbm, ones_v)
        pltpu.sync_copy(dst_hbm.at[wid], idx_v)
        plsc.subcore_barrier()

        @pl.loop(0, NCH)
        def _(j):
            pltpu.sync_copy(ones_v, acc_sh.at[idx_v.at[j]], add=True)

        plsc.subcore_barrier()
        pltpu.sync_copy(acc_sh.at[pl.ds(r0, RSUB)],
                        out_hbm.at[cid, pl.ds(r0, RSUB)])

    return deg_kernel(dst_w, zeros_nt, ones_c)


def _edge_call(hp, src_w, dst_w, d):
    @functools.partial(
        pl.kernel,
        out_type=jax.ShapeDtypeStruct((2, NT, d), jnp.float32),
        mesh=_sc_mesh(),
        compiler_params=_SC_PARAMS,
        scratch_types=[
            pltpu.VMEM((W, C), jnp.int32),
            pltpu.VMEM((W, C), jnp.int32),
            pltpu.VMEM((W, C), jnp.int32),
            pltpu.VMEM((W, C), jnp.int32),
            pltpu.VMEM((C, d), jnp.float32),
            pltpu.VMEM((C, d), jnp.float32),
            pltpu.SemaphoreType.DMA,
            pltpu.SemaphoreType.DMA,
            pltpu.SemaphoreType.DMA,
            pltpu.VMEM_SHARED((NT, d), jnp.float32),
        ],
    )
    def edge_kernel(hp_hbm, src_hbm, dst_hbm, out_hbm,
                    srcA, dstA, srcB, dstB, buf0, buf1,
                    gs0, gs1, isem, acc_sh):
        cid = lax.axis_index("c")
        sid = lax.axis_index("s")
        wid = cid * NSUB + sid
        r0 = sid * RSUB
        wins = ((srcA, dstA), (srcB, dstB))

        def fire_g(sw, jj, buf, sem):
            pltpu.async_copy(hp_hbm.at[sw.at[jj]], buf, sem)

        def wait_g(buf, sem):
            pltpu.make_async_copy(hp_hbm.at[srcA.at[0]], buf, sem).wait()

        def scat(dw, jj, buf):
            pltpu.sync_copy(buf, acc_sh.at[dw.at[jj]], add=True)

        def pre_idx(w, sw, dw):
            pltpu.async_copy(src_hbm.at[wid, pl.ds(w * W, W)], sw, isem)
            pltpu.async_copy(dst_hbm.at[wid, pl.ds(w * W, W)], dw, isem)

        def wait_idx(sw, dw):
            pltpu.make_async_copy(src_hbm.at[wid, pl.ds(0, W)], sw,
                                  isem).wait()
            pltpu.make_async_copy(src_hbm.at[wid, pl.ds(0, W)], dw,
                                  isem).wait()

        pre_idx(0, srcA, dstA)
        pltpu.sync_copy(hp_hbm.at[pl.ds(r0, RSUB)], acc_sh.at[pl.ds(r0, RSUB)])
        wait_idx(srcA, dstA)
        plsc.subcore_barrier()

        for w in range(NWIN):
            sw, dw = wins[w % 2]
            nsw, ndw = wins[(w + 1) % 2]
            if w + 1 < NWIN:
                pre_idx(w + 1, nsw, ndw)
            fire_g(sw, 0, buf0, gs0)

            @pl.loop(0, W, step=2)
            def _(j):
                fire_g(sw, j + 1, buf1, gs1)
                wait_g(buf0, gs0)
                scat(dw, j, buf0)

                @pl.when(j + 2 < W)
                def _():
                    fire_g(sw, j + 2, buf0, gs0)

                wait_g(buf1, gs1)
                scat(dw, j + 1, buf1)

            if w + 1 < NWIN:
                wait_idx(nsw, ndw)

        plsc.subcore_barrier()
        pltpu.sync_copy(acc_sh.at[pl.ds(r0, RSUB)],
                        out_hbm.at[cid, pl.ds(r0, RSUB)])

    return edge_kernel(hp, src_w, dst_w)


def _k1_body(x_ref, w_ref, degp_ref, hp_ref, dinv_ref):
    deg = degp_ref[0] + degp_ref[1] + 1.0
    dinv = jax.lax.rsqrt(deg)
    hp_ref[...] = _mm(x_ref[...], w_ref[...]) * dinv
    dinv_ref[...] = dinv


def _first_layer(xs, w1, degp):
    return pl.pallas_call(
        _k1_body,
        grid=(GRID,),
        in_specs=[
            pl.BlockSpec((RB, DIN), lambda i: (i, 0)),
            pl.BlockSpec((DIN, DH), lambda i: (0, 0)),
            pl.BlockSpec((2, RB, 1), lambda i: (0, i, 0)),
        ],
        out_specs=[
            pl.BlockSpec((RB, DH), lambda i: (i, 0)),
            pl.BlockSpec((RB, 1), lambda i: (i, 0)),
        ],
        out_shape=[
            jax.ShapeDtypeStruct((NT, DH), jnp.float32),
            jax.ShapeDtypeStruct((NT, 1), jnp.float32),
        ],
    )(xs, w1, degp)


def _mid_body(part_ref, hp_ref, dinv_ref, b_ref, w_ref, out_ref):
    dinv = dinv_ref[...]
    a = dinv * (part_ref[0] + part_ref[1] - hp_ref[...]) + b_ref[...]
    a = jnp.maximum(a, 0.0)
    out_ref[...] = _mm(a, w_ref[...]) * dinv


def _mid_layer(part, hp, dinv, b_row, w, d_in, d_out):
    return pl.pallas_call(
        _mid_body,
        grid=(GRID,),
        in_specs=[
            pl.BlockSpec((2, RB, d_in), lambda i: (0, i, 0)),
            pl.BlockSpec((RB, d_in), lambda i: (i, 0)),
            pl.BlockSpec((RB, 1), lambda i: (i, 0)),
            pl.BlockSpec((1, d_in), lambda i: (0, 0)),
            pl.BlockSpec((d_in, d_out), lambda i: (0, 0)),
        ],
        out_specs=pl.BlockSpec((RB, d_out), lambda i: (i, 0)),
        out_shape=jax.ShapeDtypeStruct((NT, d_out), jnp.float32),
    )(part, hp, dinv, b_row, w)


def _pool_body(part_ref, hp_ref, dinv_ref, b_ref, batch_ref,
               sums_ref, cnts_ref):
    i = pl.program_id(0)

    @pl.when(i == 0)
    def _():
        sums_ref[...] = jnp.zeros_like(sums_ref)
        cnts_ref[...] = jnp.zeros_like(cnts_ref)

    dinv = dinv_ref[...]
    h = dinv * (part_ref[0] + part_ref[1] - hp_ref[...]) + b_ref[...]
    gid = jax.lax.broadcasted_iota(jnp.int32, (1, NGT), 1)
    oh = (batch_ref[...] == gid).astype(jnp.float32)
    sums_ref[...] += jax.lax.dot_general(
        oh, h, (((0,), (0,)), ((), ())),
        precision=_HI, preferred_element_type=jnp.float32)
    cnts_ref[...] += jnp.sum(oh, axis=0)[:, None]


def _pool_layer(part, hp, dinv, b_row, batch_col):
    return pl.pallas_call(
        _pool_body,
        grid=(GRID,),
        in_specs=[
            pl.BlockSpec((2, RB, DE), lambda i: (0, i, 0)),
            pl.BlockSpec((RB, DE), lambda i: (i, 0)),
            pl.BlockSpec((RB, 1), lambda i: (i, 0)),
            pl.BlockSpec((1, DE), lambda i: (0, 0)),
            pl.BlockSpec((RB, 1), lambda i: (i, 0)),
        ],
        out_specs=[
            pl.BlockSpec((NGT, DE), lambda i: (0, 0)),
            pl.BlockSpec((NGT, 1), lambda i: (0, 0)),
        ],
        out_shape=[
            jax.ShapeDtypeStruct((NGT, DE), jnp.float32),
            jax.ShapeDtypeStruct((NGT, 1), jnp.float32),
        ],
    )(part, hp, dinv, b_row, batch_col)


def _head_body(sums_ref, cnts_ref, comp_ref, wc_ref, bc_ref,
               wa_ref, wb_ref, wcf_ref, bf1_ref, wf2_ref, bf2_ref,
               wf3_ref, bf3_ref, out_ref):
    emb = sums_ref[...] / jnp.maximum(cnts_ref[...], 1.0)
    e1 = emb[:NG]
    e2 = emb[NG:]
    cf = jnp.maximum(_mm(comp_ref[...], wc_ref[...]) + bc_ref[...], 0.0)
    z = _mm(e1, wa_ref[...]) + _mm(e2, wb_ref[...]) + _mm(cf, wcf_ref[...])
    z = jnp.maximum(z + bf1_ref[...], 0.0)
    z = jnp.maximum(_mm(z, wf2_ref[...]) + bf2_ref[...], 0.0)
    z = _mm(z, wf3_ref[...]) + bf3_ref[...]
    out_ref[...] = jax.nn.sigmoid(z)


def _head(sums, cnts, comp, wc, bc, wa, wb, wcf, bf1, wf2, bf2, wf3, bf3):
    return pl.pallas_call(
        _head_body,
        out_shape=jax.ShapeDtypeStruct((NG, 1), jnp.float32),
    )(sums, cnts, comp, wc, bc, wa, wb, wcf, bf1, wf2, bf2, wf3, bf3)


def kernel(x1, x2, comp_features, edge_index1, edge_index2, batch1, batch2,
           W1, b1, W2, b2, W3, b3, Wc, bc, Wf1, bf1, Wf2, bf2, Wf3, bf3):
    f32 = jnp.float32
    i32 = jnp.int32

    zrows = jnp.zeros((NPAD - N, DIN), f32)
    xs = jnp.concatenate([x1, zrows, x2, zrows])

    epad = jnp.full((EP - ETOT,), DUMMY, i32)
    src = jnp.concatenate([edge_index1[0], edge_index2[0] + NPAD, epad])
    dst = jnp.concatenate([edge_index1[1], edge_index2[1] + NPAD, epad])
    src_w = src.reshape(NW, NCH, C)
    dst_w = dst.reshape(NW, NCH, C)

    bpad = jnp.full((NPAD - N,), NGT + 7, i32)
    batch_col = jnp.concatenate(
        [batch1, bpad, batch2 + NG, bpad]).reshape(NT, 1)

    zeros_nt = jnp.zeros((NT, DGW), f32)
    ones_c = jnp.ones((C, DGW), f32)

    degp = _deg_call(dst_w, zeros_nt, ones_c)[:, :, :1]

    hp1, dinv = _first_layer(xs, W1, degp)
    p1 = _edge_call(hp1, src_w, dst_w, DH)
    hp2 = _mid_layer(p1, hp1, dinv, b1.reshape(1, DH), W2, DH, DH)
    p2 = _edge_call(hp2, src_w, dst_w, DH)
    hp3 = _mid_layer(p2, hp2, dinv, b2.reshape(1, DH), W3, DH, DE)
    p3 = _edge_call(hp3, src_w, dst_w, DE)
    sums, cnts = _pool_layer(p3, hp3, dinv, b3.reshape(1, DE), batch_col)

    return _head(sums, cnts, comp_features,
                 Wc, bc.reshape(1, 16),
                 Wf1[:DE], Wf1[DE:2 * DE], Wf1[2 * DE:],
                 bf1.reshape(1, DH), Wf2, bf2.reshape(1, 32),
                 Wf3, bf3.reshape(1, 1))


# speedup vs baseline: 2.3021x; 2.3021x over previous
import functools

import jax
import jax.numpy as jnp
from jax import lax
from jax.experimental import pallas as pl
from jax.experimental.pallas import tpu as pltpu
from jax.experimental.pallas import tpu_sc as plsc

N = 10000
NG = 64
NGT = 2 * NG
DIN = 128
DH = 64
DE = 32
E = 320000

NPAD = 10240
NT = 2 * NPAD
NSUB = 16
RSUB = NT // NSUB
DGW = 16
C = 128
NW = 32
ETOT = 2 * E
W = 32
NCH = ((-(-ETOT // (NW * C)) + W - 1) // W) * W
NWIN = NCH // W
EP = NW * C * NCH

RB = 1280
GRID = NT // RB

_SC_PARAMS = pltpu.CompilerParams(use_tc_tiling_on_sc=False)


def _sc_mesh():
    return plsc.VectorSubcoreMesh(core_axis_name="c", subcore_axis_name="s",
                                  num_cores=2, num_subcores=NSUB)

_HI = jax.lax.Precision.HIGHEST


def _mm(a, b):
    return jax.lax.dot_general(a, b, (((1,), (0,)), ((), ())),
                               precision=_HI,
                               preferred_element_type=jnp.float32)


def _deg_call(dst_w, zeros_nt, ones_c):
    @functools.partial(
        pl.kernel,
        out_type=jax.ShapeDtypeStruct((2, NT, DGW), jnp.float32),
        mesh=_sc_mesh(),
        compiler_params=_SC_PARAMS,
        scratch_types=[
            pltpu.VMEM((NCH, C), jnp.int32),
            pltpu.VMEM((C, DGW), jnp.float32),
            pltpu.VMEM_SHARED((NT, DGW), jnp.float32),
        ],
    )
    def deg_kernel(dst_hbm, zero_hbm, one_hbm, out_hbm, idx_v, ones_v, acc_sh):
        cid = lax.axis_index("c")
        sid = lax.axis_index("s")
        wid = cid * NSUB + sid
        r0 = sid * RSUB
        pltpu.sync_copy(zero_hbm.at[pl.ds(r0, RSUB)], acc_sh.at[pl.ds(r0, RSUB)])
        pltpu.sync_copy(one_hbm, ones_v)
        pltpu.sync_copy(dst_hbm.at[wid], idx_v)
        plsc.subcore_barrier()

        @pl.loop(0, NCH)
        def _(j):
            pltpu.sync_copy(ones_v, acc_sh.at[idx_v.at[j]], add=True)

        plsc.subcore_barrier()
        pltpu.sync_copy(acc_sh.at[pl.ds(r0, RSUB)],
                        out_hbm.at[cid, pl.ds(r0, RSUB)])

    return deg_kernel(dst_w, zeros_nt, ones_c)


def _edge_call(hp, src_w, dst_w, d):
    @functools.partial(
        pl.kernel,
        out_type=jax.ShapeDtypeStruct((2, NT, d), jnp.float32),
        mesh=_sc_mesh(),
        compiler_params=_SC_PARAMS,
        scratch_types=[
            pltpu.VMEM((W, C), jnp.int32),
            pltpu.VMEM((W, C), jnp.int32),
            pltpu.VMEM((W, C), jnp.int32),
            pltpu.VMEM((W, C), jnp.int32),
            pltpu.VMEM((C, d), jnp.float32),
            pltpu.VMEM((C, d), jnp.float32),
            pltpu.SemaphoreType.DMA,
            pltpu.SemaphoreType.DMA,
            pltpu.SemaphoreType.DMA,
            pltpu.VMEM_SHARED((NT, d), jnp.float32),
        ],
    )
    def edge_kernel(hp_hbm, src_hbm, dst_hbm, out_hbm,
                    srcA, dstA, srcB, dstB, buf0, buf1,
                    gs0, gs1, isem, acc_sh):
        cid = lax.axis_index("c")
        sid = lax.axis_index("s")
        wid = cid * NSUB + sid
        r0 = sid * RSUB
        wins = ((srcA, dstA), (srcB, dstB))

        def fire_g(sw, jj, buf, sem):
            pltpu.async_copy(hp_hbm.at[sw.at[jj]], buf, sem)

        def wait_g(buf, sem):
            pltpu.make_async_copy(hp_hbm.at[srcA.at[0]], buf, sem).wait()

        def scat(dw, jj, buf):
            pltpu.sync_copy(buf, acc_sh.at[dw.at[jj]], add=True)

        def pre_idx(w, sw, dw):
            pltpu.async_copy(src_hbm.at[wid, pl.ds(w * W, W)], sw, isem)
            pltpu.async_copy(dst_hbm.at[wid, pl.ds(w * W, W)], dw, isem)

        def wait_idx(sw, dw):
            pltpu.make_async_copy(src_hbm.at[wid, pl.ds(0, W)], sw,
                                  isem).wait()
            pltpu.make_async_copy(src_hbm.at[wid, pl.ds(0, W)], dw,
                                  isem).wait()

        pre_idx(0, srcA, dstA)
        pltpu.sync_copy(hp_hbm.at[pl.ds(r0, RSUB)], acc_sh.at[pl.ds(r0, RSUB)])
        wait_idx(srcA, dstA)
        plsc.subcore_barrier()

        for w in range(NWIN):
            sw, dw = wins[w % 2]
            nsw, ndw = wins[(w + 1) % 2]
            if w + 1 < NWIN:
                pre_idx(w + 1, nsw, ndw)
            fire_g(sw, 0, buf0, gs0)

            @pl.loop(0, W, step=2)
            def _(j):
                fire_g(sw, j + 1, buf1, gs1)
                wait_g(buf0, gs0)
                scat(dw, j, buf0)

                @pl.when(j + 2 < W)
                def _():
                    fire_g(sw, j + 2, buf0, gs0)

                wait_g(buf1, gs1)
                scat(dw, j + 1, buf1)

            if w + 1 < NWIN:
                wait_idx(nsw, ndw)

        plsc.subcore_barrier()
        pltpu.sync_copy(acc_sh.at[pl.ds(r0, RSUB)],
                        out_hbm.at[cid, pl.ds(r0, RSUB)])

    return edge_kernel(hp, src_w, dst_w)


def _k1_body(x_ref, w_ref, degp_ref, hp_ref, dinv_ref):
    deg = degp_ref[0] + degp_ref[1] + 1.0
    dinv = jax.lax.rsqrt(deg)
    hp_ref[...] = _mm(x_ref[...], w_ref[...]) * dinv
    dinv_ref[...] = dinv


def _first_layer(xs, w1, degp):
    return pl.pallas_call(
        _k1_body,
        grid=(GRID,),
        in_specs=[
            pl.BlockSpec((RB, DIN), lambda i: (i, 0)),
            pl.BlockSpec((DIN, DH), lambda i: (0, 0)),
            pl.BlockSpec((2, RB, 1), lambda i: (0, i, 0)),
        ],
        out_specs=[
            pl.BlockSpec((RB, DH), lambda i: (i, 0)),
            pl.BlockSpec((RB, 1), lambda i: (i, 0)),
        ],
        out_shape=[
            jax.ShapeDtypeStruct((NT, DH), jnp.float32),
            jax.ShapeDtypeStruct((NT, 1), jnp.float32),
        ],
    )(xs, w1, degp)


def _mid_body(part_ref, hp_ref, dinv_ref, b_ref, w_ref, out_ref):
    dinv = dinv_ref[...]
    a = dinv * (part_ref[0] + part_ref[1] - hp_ref[...]) + b_ref[...]
    a = jnp.maximum(a, 0.0)
    out_ref[...] = _mm(a, w_ref[...]) * dinv


def _mid_layer(part, hp, dinv, b_row, w, d_in, d_out):
    return pl.pallas_call(
        _mid_body,
        grid=(GRID,),
        in_specs=[
            pl.BlockSpec((2, RB, d_in), lambda i: (0, i, 0)),
            pl.BlockSpec((RB, d_in), lambda i: (i, 0)),
            pl.BlockSpec((RB, 1), lambda i: (i, 0)),
            pl.BlockSpec((1, d_in), lambda i: (0, 0)),
            pl.BlockSpec((d_in, d_out), lambda i: (0, 0)),
        ],
        out_specs=pl.BlockSpec((RB, d_out), lambda i: (i, 0)),
        out_shape=jax.ShapeDtypeStruct((NT, d_out), jnp.float32),
    )(part, hp, dinv, b_row, w)


def _pool_body(part_ref, hp_ref, dinv_ref, b_ref, batch_ref,
               sums_ref, cnts_ref):
    i = pl.program_id(0)

    @pl.when(i == 0)
    def _():
        sums_ref[...] = jnp.zeros_like(sums_ref)
        cnts_ref[...] = jnp.zeros_like(cnts_ref)

    dinv = dinv_ref[...]
    h = dinv * (part_ref[0] + part_ref[1] - hp_ref[...]) + b_ref[...]
    gid = jax.lax.broadcasted_iota(jnp.int32, (1, NGT), 1)
    oh = (batch_ref[...] == gid).astype(jnp.float32)
    sums_ref[...] += jax.lax.dot_general(
        oh, h, (((0,), (0,)), ((), ())),
        precision=_HI, preferred_element_type=jnp.float32)
    cnts_ref[...] += jnp.sum(oh, axis=0)[:, None]


def _pool_layer(part, hp, dinv, b_row, batch_col):
    return pl.pallas_call(
        _pool_body,
        grid=(GRID,),
        in_specs=[
            pl.BlockSpec((2, RB, DE), lambda i: (0, i, 0)),
            pl.BlockSpec((RB, DE), lambda i: (i, 0)),
            pl.BlockSpec((RB, 1), lambda i: (i, 0)),
            pl.BlockSpec((1, DE), lambda i: (0, 0)),
            pl.BlockSpec((RB, 1), lambda i: (i, 0)),
        ],
        out_specs=[
            pl.BlockSpec((NGT, DE), lambda i: (0, 0)),
            pl.BlockSpec((NGT, 1), lambda i: (0, 0)),
        ],
        out_shape=[
            jax.ShapeDtypeStruct((NGT, DE), jnp.float32),
            jax.ShapeDtypeStruct((NGT, 1), jnp.float32),
        ],
    )(part, hp, dinv, b_row, batch_col)


def _head_body(sums_ref, cnts_ref, comp_ref, wc_ref, bc_ref,
               wa_ref, wb_ref, wcf_ref, bf1_ref, wf2_ref, bf2_ref,
               wf3_ref, bf3_ref, out_ref):
    emb = sums_ref[...] / jnp.maximum(cnts_ref[...], 1.0)
    e1 = emb[:NG]
    e2 = emb[NG:]
    cf = jnp.maximum(_mm(comp_ref[...], wc_ref[...]) + bc_ref[...], 0.0)
    z = _mm(e1, wa_ref[...]) + _mm(e2, wb_ref[...]) + _mm(cf, wcf_ref[...])
    z = jnp.maximum(z + bf1_ref[...], 0.0)
    z = jnp.maximum(_mm(z, wf2_ref[...]) + bf2_ref[...], 0.0)
    z = _mm(z, wf3_ref[...]) + bf3_ref[...]
    out_ref[...] = jax.nn.sigmoid(z)


def _head(sums, cnts, comp, wc, bc, wa, wb, wcf, bf1, wf2, bf2, wf3, bf3):
    return pl.pallas_call(
        _head_body,
        out_shape=jax.ShapeDtypeStruct((NG, 1), jnp.float32),
    )(sums, cnts, comp, wc, bc, wa, wb, wcf, bf1, wf2, bf2, wf3, bf3)


def kernel(x1, x2, comp_features, edge_index1, edge_index2, batch1, batch2,
           W1, b1, W2, b2, W3, b3, Wc, bc, Wf1, bf1, Wf2, bf2, Wf3, bf3):
    f32 = jnp.float32
    i32 = jnp.int32

    zrows = jnp.zeros((NPAD - N, DIN), f32)
    xs = jnp.concatenate([x1, zrows, x2, zrows])

    epad = N + (jnp.arange(EP - ETOT, dtype=i32) % (NPAD - N))
    src = jnp.concatenate([edge_index1[0], edge_index2[0] + NPAD, epad])
    dst = jnp.concatenate([edge_index1[1], edge_index2[1] + NPAD, epad])
    src_w = src.reshape(NW, NCH, C)
    dst_w = dst.reshape(NW, NCH, C)

    bpad = jnp.full((NPAD - N,), NGT + 7, i32)
    batch_col = jnp.concatenate(
        [batch1, bpad, batch2 + NG, bpad]).reshape(NT, 1)

    zeros_nt = jnp.zeros((NT, DGW), f32)
    ones_c = jnp.ones((C, DGW), f32)

    degp = _deg_call(dst_w, zeros_nt, ones_c)[:, :, :1]

    hp1, dinv = _first_layer(xs, W1, degp)
    p1 = _edge_call(hp1, src_w, dst_w, DH)
    hp2 = _mid_layer(p1, hp1, dinv, b1.reshape(1, DH), W2, DH, DH)
    p2 = _edge_call(hp2, src_w, dst_w, DH)
    hp3 = _mid_layer(p2, hp2, dinv, b2.reshape(1, DH), W3, DH, DE)
    p3 = _edge_call(hp3, src_w, dst_w, DE)
    sums, cnts = _pool_layer(p3, hp3, dinv, b3.reshape(1, DE), batch_col)

    return _head(sums, cnts, comp_features,
                 Wc, bc.reshape(1, 16),
                 Wf1[:DE], Wf1[DE:2 * DE], Wf1[2 * DE:],
                 bf1.reshape(1, DH), Wf2, bf2.reshape(1, 32),
                 Wf3, bf3.reshape(1, 1))


# speedup vs baseline: 2.5433x; 1.1048x over previous
import functools

import jax
import jax.numpy as jnp
from jax import lax
from jax.experimental import pallas as pl
from jax.experimental.pallas import tpu as pltpu
from jax.experimental.pallas import tpu_sc as plsc

N = 10000
NG = 64
NGT = 2 * NG
DIN = 128
DH = 64
DE = 32
E = 320000

NPAD = 10240
NSUB = 16
RSUBP = NPAD // NSUB
DGW = 16
C = 128
W = 32
NCH = ((-(-E // (NSUB * C)) + W - 1) // W) * W
NWIN = NCH // W
EPP = NSUB * C * NCH

RB = 1280
GPG = NPAD // RB
GRID = 2 * GPG

_SC_PARAMS = pltpu.CompilerParams(use_tc_tiling_on_sc=False)


def _sc_mesh():
    return plsc.VectorSubcoreMesh(core_axis_name="c", subcore_axis_name="s",
                                  num_cores=2, num_subcores=NSUB)

_HI = jax.lax.Precision.HIGHEST


def _mm(a, b):
    return jax.lax.dot_general(a, b, (((1,), (0,)), ((), ())),
                               precision=_HI,
                               preferred_element_type=jnp.float32)


def _bspec(d):
    return pl.BlockSpec((1, RB, d), lambda i: (i // GPG, i % GPG, 0))


def _deg_call(dst_w, zeros_pg, ones_c):
    @functools.partial(
        pl.kernel,
        out_type=jax.ShapeDtypeStruct((2, NPAD, DGW), jnp.float32),
        mesh=_sc_mesh(),
        compiler_params=_SC_PARAMS,
        scratch_types=[
            pltpu.VMEM((NCH, C), jnp.int32),
            pltpu.VMEM((C, DGW), jnp.float32),
            pltpu.VMEM_SHARED((NPAD, DGW), jnp.float32),
        ],
    )
    def deg_kernel(dst_hbm, zero_hbm, one_hbm, out_hbm, idx_v, ones_v, acc_sh):
        cid = lax.axis_index("c")
        sid = lax.axis_index("s")
        r0 = sid * RSUBP
        pltpu.sync_copy(zero_hbm.at[pl.ds(r0, RSUBP)],
                        acc_sh.at[pl.ds(r0, RSUBP)])
        pltpu.sync_copy(one_hbm, ones_v)
        pltpu.sync_copy(dst_hbm.at[cid, sid], idx_v)
        plsc.subcore_barrier()

        @pl.loop(0, NCH)
        def _(j):
            pltpu.sync_copy(ones_v, acc_sh.at[idx_v.at[j]], add=True)

        plsc.subcore_barrier()
        pltpu.sync_copy(acc_sh.at[pl.ds(r0, RSUBP)],
                        out_hbm.at[cid, pl.ds(r0, RSUBP)])

    return deg_kernel(dst_w, zeros_pg, ones_c)


def _edge_call(hp, src_w, dst_w, d):
    @functools.partial(
        pl.kernel,
        out_type=jax.ShapeDtypeStruct((2, NPAD, d), jnp.float32),
        mesh=_sc_mesh(),
        compiler_params=_SC_PARAMS,
        scratch_types=[
            pltpu.VMEM((W, C), jnp.int32),
            pltpu.VMEM((W, C), jnp.int32),
            pltpu.VMEM((W, C), jnp.int32),
            pltpu.VMEM((W, C), jnp.int32),
            pltpu.VMEM((C, d), jnp.float32),
            pltpu.VMEM((C, d), jnp.float32),
            pltpu.SemaphoreType.DMA,
            pltpu.SemaphoreType.DMA,
            pltpu.SemaphoreType.DMA,
            pltpu.VMEM_SHARED((NPAD, d), jnp.float32),
        ],
    )
    def edge_kernel(hp_hbm, src_hbm, dst_hbm, out_hbm,
                    srcA, dstA, srcB, dstB, buf0, buf1,
                    gs0, gs1, isem, acc_sh):
        cid = lax.axis_index("c")
        sid = lax.axis_index("s")
        r0 = sid * RSUBP
        wins = ((srcA, dstA), (srcB, dstB))
        hp_g = hp_hbm.at[cid]

        def fire_g(sw, jj, buf, sem):
            pltpu.async_copy(hp_g.at[sw.at[jj]], buf, sem)

        def wait_g(buf, sem):
            pltpu.make_async_copy(hp_g.at[srcA.at[0]], buf, sem).wait()

        def scat(dw, jj, buf):
            pltpu.sync_copy(buf, acc_sh.at[dw.at[jj]], add=True)

        def pre_idx(w, sw, dw):
            pltpu.async_copy(src_hbm.at[cid, sid, pl.ds(w * W, W)], sw, isem)
            pltpu.async_copy(dst_hbm.at[cid, sid, pl.ds(w * W, W)], dw, isem)

        def wait_idx(sw, dw):
            pltpu.make_async_copy(src_hbm.at[cid, sid, pl.ds(0, W)], sw,
                                  isem).wait()
            pltpu.make_async_copy(src_hbm.at[cid, sid, pl.ds(0, W)], dw,
                                  isem).wait()

        pre_idx(0, srcA, dstA)
        pltpu.sync_copy(hp_g.at[pl.ds(r0, RSUBP)], acc_sh.at[pl.ds(r0, RSUBP)])
        wait_idx(srcA, dstA)
        plsc.subcore_barrier()

        for w in range(NWIN):
            sw, dw = wins[w % 2]
            nsw, ndw = wins[(w + 1) % 2]
            if w + 1 < NWIN:
                pre_idx(w + 1, nsw, ndw)
            fire_g(sw, 0, buf0, gs0)

            @pl.loop(0, W, step=2)
            def _(j):
                fire_g(sw, j + 1, buf1, gs1)
                wait_g(buf0, gs0)
                scat(dw, j, buf0)

                @pl.when(j + 2 < W)
                def _():
                    fire_g(sw, j + 2, buf0, gs0)

                wait_g(buf1, gs1)
                scat(dw, j + 1, buf1)

            if w + 1 < NWIN:
                wait_idx(nsw, ndw)

        plsc.subcore_barrier()
        pltpu.sync_copy(acc_sh.at[pl.ds(r0, RSUBP)],
                        out_hbm.at[cid, pl.ds(r0, RSUBP)])

    return edge_kernel(hp, src_w, dst_w)


def _mm1_body(x_ref, w_ref, mm_ref):
    mm_ref[0] = _mm(x_ref[0], w_ref[...])


def _mm1(xst, w1):
    return pl.pallas_call(
        _mm1_body,
        grid=(GRID,),
        in_specs=[
            _bspec(DIN),
            pl.BlockSpec((DIN, DH), lambda i: (0, 0)),
        ],
        out_specs=_bspec(DH),
        out_shape=jax.ShapeDtypeStruct((2, NPAD, DH), jnp.float32),
    )(xst, w1)


def _scale_body(mm_ref, degp_ref, hp_ref, dinv_ref):
    deg = degp_ref[0] + 1.0
    dinv = jax.lax.rsqrt(deg)
    hp_ref[0] = mm_ref[0] * dinv
    dinv_ref[0] = dinv


def _scale(mm1, degp):
    return pl.pallas_call(
        _scale_body,
        grid=(GRID,),
        in_specs=[_bspec(DH), _bspec(1)],
        out_specs=[_bspec(DH), _bspec(1)],
        out_shape=[
            jax.ShapeDtypeStruct((2, NPAD, DH), jnp.float32),
            jax.ShapeDtypeStruct((2, NPAD, 1), jnp.float32),
        ],
    )(mm1, degp)


def _mid_body(part_ref, dinv_ref, b_ref, w_ref, out_ref):
    dinv = dinv_ref[0]
    a = jnp.maximum(dinv * part_ref[0] + b_ref[...], 0.0)
    out_ref[0] = _mm(a, w_ref[...]) * dinv


def _mid_layer(part, dinv, b_row, w, d_in, d_out):
    return pl.pallas_call(
        _mid_body,
        grid=(GRID,),
        in_specs=[
            _bspec(d_in),
            _bspec(1),
            pl.BlockSpec((1, d_in), lambda i: (0, 0)),
            pl.BlockSpec((d_in, d_out), lambda i: (0, 0)),
        ],
        out_specs=_bspec(d_out),
        out_shape=jax.ShapeDtypeStruct((2, NPAD, d_out), jnp.float32),
    )(part, dinv, b_row, w)


def _pool_body(part_ref, dinv_ref, b_ref, batch_ref, sums_ref, cnts_ref):
    i = pl.program_id(0)

    @pl.when(i == 0)
    def _():
        sums_ref[...] = jnp.zeros_like(sums_ref)
        cnts_ref[...] = jnp.zeros_like(cnts_ref)

    h = dinv_ref[0] * part_ref[0] + b_ref[...]
    gid = jax.lax.broadcasted_iota(jnp.int32, (1, NGT), 1)
    oh = (batch_ref[0] == gid).astype(jnp.float32)
    sums_ref[...] += jax.lax.dot_general(
        oh, h, (((0,), (0,)), ((), ())),
        precision=_HI, preferred_element_type=jnp.float32)
    cnts_ref[...] += jnp.sum(oh, axis=0)[:, None]


def _pool_layer(part, dinv, b_row, batch_st):
    return pl.pallas_call(
        _pool_body,
        grid=(GRID,),
        in_specs=[
            _bspec(DE),
            _bspec(1),
            pl.BlockSpec((1, DE), lambda i: (0, 0)),
            _bspec(1),
        ],
        out_specs=[
            pl.BlockSpec((NGT, DE), lambda i: (0, 0)),
            pl.BlockSpec((NGT, 1), lambda i: (0, 0)),
        ],
        out_shape=[
            jax.ShapeDtypeStruct((NGT, DE), jnp.float32),
            jax.ShapeDtypeStruct((NGT, 1), jnp.float32),
        ],
    )(part, dinv, b_row, batch_st)


def _head_body(sums_ref, cnts_ref, comp_ref, wc_ref, bc_ref,
               wa_ref, wb_ref, wcf_ref, bf1_ref, wf2_ref, bf2_ref,
               wf3_ref, bf3_ref, out_ref):
    emb = sums_ref[...] / jnp.maximum(cnts_ref[...], 1.0)
    e1 = emb[:NG]
    e2 = emb[NG:]
    cf = jnp.maximum(_mm(comp_ref[...], wc_ref[...]) + bc_ref[...], 0.0)
    z = _mm(e1, wa_ref[...]) + _mm(e2, wb_ref[...]) + _mm(cf, wcf_ref[...])
    z = jnp.maximum(z + bf1_ref[...], 0.0)
    z = jnp.maximum(_mm(z, wf2_ref[...]) + bf2_ref[...], 0.0)
    z = _mm(z, wf3_ref[...]) + bf3_ref[...]
    out_ref[...] = jax.nn.sigmoid(z)


def _head(sums, cnts, comp, wc, bc, wa, wb, wcf, bf1, wf2, bf2, wf3, bf3):
    return pl.pallas_call(
        _head_body,
        out_shape=jax.ShapeDtypeStruct((NG, 1), jnp.float32),
    )(sums, cnts, comp, wc, bc, wa, wb, wcf, bf1, wf2, bf2, wf3, bf3)


def kernel(x1, x2, comp_features, edge_index1, edge_index2, batch1, batch2,
           W1, b1, W2, b2, W3, b3, Wc, bc, Wf1, bf1, Wf2, bf2, Wf3, bf3):
    f32 = jnp.float32
    i32 = jnp.int32

    epad = N + (jnp.arange(EPP - E, dtype=i32) % (NPAD - N))

    def prep(e):
        return jnp.concatenate([e, epad]).reshape(NSUB, NCH, C)

    src_w = jnp.stack([prep(edge_index1[0]), prep(edge_index2[0])])
    dst_w = jnp.stack([prep(edge_index1[1]), prep(edge_index2[1])])

    zeros_pg = jnp.zeros((NPAD, DGW), f32)
    ones_c = jnp.ones((C, DGW), f32)
    degp = _deg_call(dst_w, zeros_pg, ones_c)[:, :, :1]

    zrows = jnp.zeros((NPAD - N, DIN), f32)
    xst = jnp.stack([jnp.concatenate([x1, zrows]),
                     jnp.concatenate([x2, zrows])])
    mm1 = _mm1(xst, W1)

    bpad = jnp.full((NPAD - N,), NGT + 7, i32)
    batch_st = jnp.stack([jnp.concatenate([batch1, bpad]),
                          jnp.concatenate([batch2 + NG, bpad])])
    batch_st = batch_st.reshape(2, NPAD, 1)

    hp1, dinv = _scale(mm1, degp)
    p1 = _edge_call(hp1, src_w, dst_w, DH)
    hp2 = _mid_layer(p1, dinv, b1.reshape(1, DH), W2, DH, DH)
    p2 = _edge_call(hp2, src_w, dst_w, DH)
    hp3 = _mid_layer(p2, dinv, b2.reshape(1, DH), W3, DH, DE)
    p3 = _edge_call(hp3, src_w, dst_w, DE)
    sums, cnts = _pool_layer(p3, dinv, b3.reshape(1, DE), batch_st)

    return _head(sums, cnts, comp_features,
                 Wc, bc.reshape(1, 16),
                 Wf1[:DE], Wf1[DE:2 * DE], Wf1[2 * DE:],
                 bf1.reshape(1, DH), Wf2, bf2.reshape(1, 32),
                 Wf3, bf3.reshape(1, 1))


# speedup vs baseline: 3.1678x; 1.2455x over previous
import functools

import jax
import jax.numpy as jnp
from jax import lax
from jax.experimental import pallas as pl
from jax.experimental.pallas import tpu as pltpu
from jax.experimental.pallas import tpu_sc as plsc

N = 10000
NG = 64
NGT = 2 * NG
DIN = 128
DH = 64
DE = 32
E = 320000

NPAD = 10240
NSUB = 16
RSUBP = NPAD // NSUB
DGW = 16
C = 128
W = 32
NCH = ((-(-E // (NSUB * C)) + W - 1) // W) * W
NWIN = NCH // W
EPP = NSUB * C * NCH

RB = 1280
GPG = NPAD // RB
GRID = 2 * GPG

_SC_PARAMS = pltpu.CompilerParams(use_tc_tiling_on_sc=False)


def _sc_mesh():
    return plsc.VectorSubcoreMesh(core_axis_name="c", subcore_axis_name="s",
                                  num_cores=2, num_subcores=NSUB)

_HI = jax.lax.Precision.HIGHEST


def _mm(a, b):
    return jax.lax.dot_general(a, b, (((1,), (0,)), ((), ())),
                               precision=_HI,
                               preferred_element_type=jnp.float32)


def _bspec(d):
    return pl.BlockSpec((1, RB, d), lambda i: (i // GPG, i % GPG, 0))


def _deg_call(dst_w, zeros_pg, ones_c):
    @functools.partial(
        pl.kernel,
        out_type=jax.ShapeDtypeStruct((2, NPAD, DGW), jnp.float32),
        mesh=_sc_mesh(),
        compiler_params=_SC_PARAMS,
        scratch_types=[
            pltpu.VMEM((NCH, C), jnp.int32),
            pltpu.VMEM((C, DGW), jnp.float32),
            pltpu.VMEM_SHARED((NPAD, DGW), jnp.float32),
        ],
    )
    def deg_kernel(dst_hbm, zero_hbm, one_hbm, out_hbm, idx_v, ones_v, acc_sh):
        cid = lax.axis_index("c")
        sid = lax.axis_index("s")
        r0 = sid * RSUBP
        pltpu.sync_copy(zero_hbm.at[pl.ds(r0, RSUBP)],
                        acc_sh.at[pl.ds(r0, RSUBP)])
        pltpu.sync_copy(one_hbm, ones_v)
        pltpu.sync_copy(dst_hbm.at[cid, sid], idx_v)
        plsc.subcore_barrier()

        @pl.loop(0, NCH)
        def _(j):
            pltpu.sync_copy(ones_v, acc_sh.at[idx_v.at[j]], add=True)

        plsc.subcore_barrier()
        pltpu.sync_copy(acc_sh.at[pl.ds(r0, RSUBP)],
                        out_hbm.at[cid, pl.ds(r0, RSUBP)])

    return deg_kernel(dst_w, zeros_pg, ones_c)


def _edge_call(hp, src_w, dst_w, d):
    @functools.partial(
        pl.kernel,
        out_type=jax.ShapeDtypeStruct((2, NPAD, d), jnp.float32),
        mesh=_sc_mesh(),
        compiler_params=_SC_PARAMS,
        scratch_types=[
            pltpu.VMEM((NCH, C), jnp.int32),
            pltpu.VMEM((NCH, C), jnp.int32),
            pltpu.VMEM((C, d), jnp.float32),
            pltpu.VMEM((C, d), jnp.float32),
            pltpu.VMEM((C, d), jnp.float32),
            pltpu.VMEM((C, d), jnp.float32),
            pltpu.SemaphoreType.DMA,
            pltpu.SemaphoreType.DMA,
            pltpu.SemaphoreType.DMA,
            pltpu.SemaphoreType.DMA,
            pltpu.SemaphoreType.DMA,
            pltpu.VMEM_SHARED((NPAD, d), jnp.float32),
        ],
    )
    def edge_kernel(hp_hbm, src_hbm, dst_hbm, out_hbm,
                    srcI, dstI, buf0, buf1, buf2, buf3,
                    gs0, gs1, gs2, gs3, isem, acc_sh):
        cid = lax.axis_index("c")
        sid = lax.axis_index("s")
        r0 = sid * RSUBP
        hp_g = hp_hbm.at[cid]
        bufs = (buf0, buf1, buf2, buf3)
        sems = (gs0, gs1, gs2, gs3)

        def fire_g(c, s):
            pltpu.async_copy(hp_g.at[srcI.at[c]], bufs[s], sems[s])

        def wait_g(s):
            pltpu.make_async_copy(hp_g.at[srcI.at[0]], bufs[s],
                                  sems[s]).wait()

        def scat(c, s):
            pltpu.sync_copy(bufs[s], acc_sh.at[dstI.at[c]], add=True)

        pltpu.async_copy(src_hbm.at[cid, sid], srcI, isem)
        pltpu.async_copy(dst_hbm.at[cid, sid], dstI, isem)
        pltpu.sync_copy(hp_g.at[pl.ds(r0, RSUBP)], acc_sh.at[pl.ds(r0, RSUBP)])
        pltpu.make_async_copy(src_hbm.at[cid, sid], srcI, isem).wait()
        pltpu.make_async_copy(src_hbm.at[cid, sid], dstI, isem).wait()
        plsc.subcore_barrier()

        fire_g(0, 0)
        fire_g(1, 1)
        fire_g(2, 2)

        @pl.loop(0, NCH, step=4)
        def _(c):
            for s in range(4):
                nc = c + s + 3

                @pl.when(nc < NCH)
                def _():
                    fire_g(nc, (s + 3) % 4)

                wait_g(s)
                scat(c + s, s)

        plsc.subcore_barrier()
        pltpu.sync_copy(acc_sh.at[pl.ds(r0, RSUBP)],
                        out_hbm.at[cid, pl.ds(r0, RSUBP)])

    return edge_kernel(hp, src_w, dst_w)


def _mm1_body(x_ref, w_ref, mm_ref):
    mm_ref[0] = _mm(x_ref[0], w_ref[...])


def _mm1(xst, w1):
    return pl.pallas_call(
        _mm1_body,
        grid=(GRID,),
        in_specs=[
            _bspec(DIN),
            pl.BlockSpec((DIN, DH), lambda i: (0, 0)),
        ],
        out_specs=_bspec(DH),
        out_shape=jax.ShapeDtypeStruct((2, NPAD, DH), jnp.float32),
    )(xst, w1)


def _scale_body(mm_ref, degp_ref, hp_ref, dinv_ref):
    deg = degp_ref[0] + 1.0
    dinv = jax.lax.rsqrt(deg)
    hp_ref[0] = mm_ref[0] * dinv
    dinv_ref[0] = dinv


def _scale(mm1, degp):
    return pl.pallas_call(
        _scale_body,
        grid=(GRID,),
        in_specs=[_bspec(DH), _bspec(1)],
        out_specs=[_bspec(DH), _bspec(1)],
        out_shape=[
            jax.ShapeDtypeStruct((2, NPAD, DH), jnp.float32),
            jax.ShapeDtypeStruct((2, NPAD, 1), jnp.float32),
        ],
    )(mm1, degp)


def _mid_body(part_ref, dinv_ref, b_ref, w_ref, out_ref):
    dinv = dinv_ref[0]
    a = jnp.maximum(dinv * part_ref[0] + b_ref[...], 0.0)
    out_ref[0] = _mm(a, w_ref[...]) * dinv


def _mid_layer(part, dinv, b_row, w, d_in, d_out):
    return pl.pallas_call(
        _mid_body,
        grid=(GRID,),
        in_specs=[
            _bspec(d_in),
            _bspec(1),
            pl.BlockSpec((1, d_in), lambda i: (0, 0)),
            pl.BlockSpec((d_in, d_out), lambda i: (0, 0)),
        ],
        out_specs=_bspec(d_out),
        out_shape=jax.ShapeDtypeStruct((2, NPAD, d_out), jnp.float32),
    )(part, dinv, b_row, w)


def _pool_body(part_ref, dinv_ref, b_ref, batch_ref, sums_ref, cnts_ref):
    i = pl.program_id(0)

    @pl.when(i == 0)
    def _():
        sums_ref[...] = jnp.zeros_like(sums_ref)
        cnts_ref[...] = jnp.zeros_like(cnts_ref)

    h = dinv_ref[0] * part_ref[0] + b_ref[...]
    gid = jax.lax.broadcasted_iota(jnp.int32, (1, NGT), 1)
    oh = (batch_ref[0] == gid).astype(jnp.float32)
    sums_ref[...] += jax.lax.dot_general(
        oh, h, (((0,), (0,)), ((), ())),
        precision=_HI, preferred_element_type=jnp.float32)
    cnts_ref[...] += jnp.sum(oh, axis=0)[:, None]


def _pool_layer(part, dinv, b_row, batch_st):
    return pl.pallas_call(
        _pool_body,
        grid=(GRID,),
        in_specs=[
            _bspec(DE),
            _bspec(1),
            pl.BlockSpec((1, DE), lambda i: (0, 0)),
            _bspec(1),
        ],
        out_specs=[
            pl.BlockSpec((NGT, DE), lambda i: (0, 0)),
            pl.BlockSpec((NGT, 1), lambda i: (0, 0)),
        ],
        out_shape=[
            jax.ShapeDtypeStruct((NGT, DE), jnp.float32),
            jax.ShapeDtypeStruct((NGT, 1), jnp.float32),
        ],
    )(part, dinv, b_row, batch_st)


def _head_body(sums_ref, cnts_ref, comp_ref, wc_ref, bc_ref,
               wa_ref, wb_ref, wcf_ref, bf1_ref, wf2_ref, bf2_ref,
               wf3_ref, bf3_ref, out_ref):
    emb = sums_ref[...] / jnp.maximum(cnts_ref[...], 1.0)
    e1 = emb[:NG]
    e2 = emb[NG:]
    cf = jnp.maximum(_mm(comp_ref[...], wc_ref[...]) + bc_ref[...], 0.0)
    z = _mm(e1, wa_ref[...]) + _mm(e2, wb_ref[...]) + _mm(cf, wcf_ref[...])
    z = jnp.maximum(z + bf1_ref[...], 0.0)
    z = jnp.maximum(_mm(z, wf2_ref[...]) + bf2_ref[...], 0.0)
    z = _mm(z, wf3_ref[...]) + bf3_ref[...]
    out_ref[...] = jax.nn.sigmoid(z)


def _head(sums, cnts, comp, wc, bc, wa, wb, wcf, bf1, wf2, bf2, wf3, bf3):
    return pl.pallas_call(
        _head_body,
        out_shape=jax.ShapeDtypeStruct((NG, 1), jnp.float32),
    )(sums, cnts, comp, wc, bc, wa, wb, wcf, bf1, wf2, bf2, wf3, bf3)


def kernel(x1, x2, comp_features, edge_index1, edge_index2, batch1, batch2,
           W1, b1, W2, b2, W3, b3, Wc, bc, Wf1, bf1, Wf2, bf2, Wf3, bf3):
    f32 = jnp.float32
    i32 = jnp.int32

    epad = N + (jnp.arange(EPP - E, dtype=i32) % (NPAD - N))

    def prep(e):
        return jnp.concatenate([e, epad]).reshape(NSUB, NCH, C)

    src_w = jnp.stack([prep(edge_index1[0]), prep(edge_index2[0])])
    dst_w = jnp.stack([prep(edge_index1[1]), prep(edge_index2[1])])

    zeros_pg = jnp.zeros((NPAD, DGW), f32)
    ones_c = jnp.ones((C, DGW), f32)
    degp = _deg_call(dst_w, zeros_pg, ones_c)[:, :, :1]

    zrows = jnp.zeros((NPAD - N, DIN), f32)
    xst = jnp.stack([jnp.concatenate([x1, zrows]),
                     jnp.concatenate([x2, zrows])])
    mm1 = _mm1(xst, W1)

    bpad = jnp.full((NPAD - N,), NGT + 7, i32)
    batch_st = jnp.stack([jnp.concatenate([batch1, bpad]),
                          jnp.concatenate([batch2 + NG, bpad])])
    batch_st = batch_st.reshape(2, NPAD, 1)

    hp1, dinv = _scale(mm1, degp)
    p1 = _edge_call(hp1, src_w, dst_w, DH)
    hp2 = _mid_layer(p1, dinv, b1.reshape(1, DH), W2, DH, DH)
    p2 = _edge_call(hp2, src_w, dst_w, DH)
    hp3 = _mid_layer(p2, dinv, b2.reshape(1, DH), W3, DH, DE)
    p3 = _edge_call(hp3, src_w, dst_w, DE)
    sums, cnts = _pool_layer(p3, dinv, b3.reshape(1, DE), batch_st)

    return _head(sums, cnts, comp_features,
                 Wc, bc.reshape(1, 16),
                 Wf1[:DE], Wf1[DE:2 * DE], Wf1[2 * DE:],
                 bf1.reshape(1, DH), Wf2, bf2.reshape(1, 32),
                 Wf3, bf3.reshape(1, 1))
